# baseline (speedup 1.0000x reference)
import functools

import jax
import jax.numpy as jnp
from jax import lax
from jax.experimental import pallas as pl
from jax.experimental.pallas import tpu as pltpu
from jax.experimental.pallas import tpu_sc as plsc

_D = 128
_SCALE = float(_D) ** 0.5
_BATCH, _SEQ = 4096, 200

_R = 10000
_B1_BASE = 20000
_B2_BASE = 100000
_G2 = 60000
_TBL_ROWS = _B2_BASE + 16 * _G2
_S0 = _B1_BASE // _R
_S1 = _S0 + 4 * (80000 // 4 // _R)
_NB1 = 20000 // _R
_NB2 = _G2 // _R
_NSTEPS = _S1 + 16 * _NB2


def _table_body(w0, w1p, w2p, q, out):
    s = pl.program_id(0)
    dn = (((1,), (0,)), ((), ()))

    @pl.when(s < _S0)
    def _():
        out[...] = lax.dot_general(
            w0[...], q[0], dn, preferred_element_type=jnp.float32)

    @pl.when((s >= _S0) & (s < _S1))
    def _():
        out[...] = lax.dot_general(
            w1p[...], q[0], dn, preferred_element_type=jnp.float32)

    @pl.when(s >= _S1)
    def _():
        out[...] = lax.dot_general(
            w2p[...], q[0], dn, preferred_element_type=jnp.float32)


def _build_table(W0, W1, W2, P0, P1, P2):
    bf = jnp.bfloat16
    w0 = W0.astype(bf)
    w1p = W1.astype(bf).reshape(20000, 128)
    w2p = jnp.pad(W2.astype(bf).reshape(56250, 128),
                  ((0, _G2 - 56250), (0, 0)))

    qs = [(P0.T * _SCALE).astype(bf)]
    p1t = (P1.T * _SCALE).astype(bf)
    for j in range(4):
        qs.append(jnp.zeros((128, 128), bf)
                  .at[32 * j:32 * (j + 1), :].set(p1t))
    p2t = (P2.T * _SCALE).astype(bf)
    for j in range(16):
        qs.append(jnp.zeros((128, 128), bf)
                  .at[8 * j:8 * (j + 1), :].set(p2t))
    q = jnp.stack(qs)

    return pl.pallas_call(
        _table_body,
        grid=(_NSTEPS,),
        in_specs=[
            pl.BlockSpec((_R, 128), lambda s: (jnp.where(s < _S0, s, 0), 0)),
            pl.BlockSpec(
                (_R, 128),
                lambda s: (jnp.where((s >= _S0) & (s < _S1),
                                     (s - _S0) // 4, 0), 0)),
            pl.BlockSpec(
                (_R, 128),
                lambda s: (jnp.where(s >= _S1, (s - _S1) // 16, 0), 0)),
            pl.BlockSpec(
                (1, 128, 128),
                lambda s: (jnp.where(
                    s < _S0, 0,
                    jnp.where(s < _S1,
                              1 + (s - _S0) % 4,
                              5 + (s - _S1) % 16)), 0, 0)),
        ],
        out_specs=pl.BlockSpec(
            (_R, _D),
            lambda s: (jnp.where(
                s < _S0, s,
                jnp.where(
                    s < _S1,
                    _S0 + ((s - _S0) % 4) * _NB1 + (s - _S0) // 4,
                    _S1 + ((s - _S1) % 16) * _NB2 + (s - _S1) // 16)), 0)),
        out_shape=jax.ShapeDtypeStruct((_TBL_ROWS, _D), jnp.float32),
    )(w0, w1p, w2p, q)


_N = _BATCH * _SEQ
_NC, _NS = 2, 16
_NW = _NC * _NS
_PER_W = _N // _NW
_CH = 128
_NCH = _PER_W // _CH

_sc_mesh = plsc.VectorSubcoreMesh(core_axis_name="c", subcore_axis_name="s")


@functools.partial(
    pl.kernel,
    mesh=_sc_mesh,
    out_type=jax.ShapeDtypeStruct((_N, _D), jnp.float32),
    scratch_types=[
        pltpu.VMEM((_PER_W,), jnp.int32),
        pltpu.VMEM((_CH, _D), jnp.float32),
        pltpu.SemaphoreType.DMA,
    ],
)
def _sc_gather(ids_hbm, table_hbm, out_hbm, idx_v, rows_v, sem):
    wid = lax.axis_index("s") * _NC + lax.axis_index("c")

    pltpu.sync_copy(ids_hbm.at[pl.ds(wid * _PER_W, _PER_W)], idx_v)

    def remap(c, carry):
        base = pl.multiple_of(c * _CH, _CH)
        for k in range(_CH // 16):
            v = idx_v[pl.ds(base + k * 16, 16)]
            u1 = v - _B1_BASE
            p1 = _B1_BASE + (u1 & 3) * 20000 + (u1 >> 2)
            u2 = v - _B2_BASE
            p2 = _B2_BASE + (u2 & 15) * _G2 + (u2 >> 4)
            idx_v[pl.ds(base + k * 16, 16)] = jnp.where(
                v >= _B2_BASE, p2, jnp.where(v >= _B1_BASE, p1, v))
        return carry

    lax.fori_loop(0, _NCH, remap, 0)

    def body(c, carry):
        off = pl.multiple_of(c * _CH, _CH)
        pltpu.async_copy(
            table_hbm.at[idx_v.at[pl.ds(off, _CH)]], rows_v, sem).wait()
        pltpu.sync_copy(
            rows_v, out_hbm.at[pl.ds(pl.multiple_of(wid * _PER_W, _CH) + off,
                                     _CH)])
        return carry

    lax.fori_loop(0, _NCH, body, 0)


def kernel(input_, W0, W1, W2, P0, P1, P2):
    table = _build_table(W0, W1, W2, P0, P1, P2)
    ids = input_.reshape(_N)
    out = _sc_gather(ids, table)
    return out.reshape(_BATCH, _SEQ, _D)


# speedup vs baseline: 1.2828x; 1.2828x over previous
import functools

import jax
import jax.numpy as jnp
from jax import lax
from jax.experimental import pallas as pl
from jax.experimental.pallas import tpu as pltpu
from jax.experimental.pallas import tpu_sc as plsc

_D = 128
_SCALE = float(_D) ** 0.5
_BATCH, _SEQ = 4096, 200
_NUM_TOKENS = 1000000

_R = 20000
_NSTEPS = _NUM_TOKENS // _R
_S0 = 20000 // _R
_S1 = 100000 // _R


def _table_body(w0, w1, w2, q0, q1, q2, out):
    s = pl.program_id(0)
    dn = (((1,), (0,)), ((), ()))

    @pl.when(s < _S0)
    def _():
        out[...] = lax.dot_general(
            w0[...], q0[...], dn, preferred_element_type=jnp.float32)

    @pl.when((s >= _S0) & (s < _S1))
    def _():
        out[...] = lax.dot_general(
            w1[...], q1[...], dn, preferred_element_type=jnp.float32)

    @pl.when(s >= _S1)
    def _():
        out[...] = lax.dot_general(
            w2[...], q2[...], dn, preferred_element_type=jnp.float32)


def _build_table(W0, W1, W2, P0, P1, P2):
    bf = jnp.bfloat16
    return pl.pallas_call(
        _table_body,
        grid=(_NSTEPS,),
        in_specs=[
            pl.BlockSpec((_R, 128), lambda s: (jnp.where(s < _S0, s, 0), 0)),
            pl.BlockSpec((_R, 32),
                         lambda s: (jnp.clip(s - _S0, 0, _S1 - _S0 - 1), 0)),
            pl.BlockSpec((_R, 8),
                         lambda s: (jnp.clip(s - _S1, 0,
                                             _NSTEPS - _S1 - 1), 0)),
            pl.BlockSpec((128, 128), lambda s: (0, 0)),
            pl.BlockSpec((32, 128), lambda s: (0, 0)),
            pl.BlockSpec((8, 128), lambda s: (0, 0)),
        ],
        out_specs=pl.BlockSpec((_R, _D), lambda s: (s, 0)),
        out_shape=jax.ShapeDtypeStruct((_NUM_TOKENS, _D), jnp.float32),
    )(W0.astype(bf), W1.astype(bf), W2.astype(bf),
      (P0.T * _SCALE).astype(bf), (P1.T * _SCALE).astype(bf),
      (P2.T * _SCALE).astype(bf))


_N = _BATCH * _SEQ
_NC, _NS = 2, 16
_NW = _NC * _NS
_PER_W = _N // _NW
_CH = 128
_NCH = _PER_W // _CH

_sc_mesh = plsc.VectorSubcoreMesh(core_axis_name="c", subcore_axis_name="s")


@functools.partial(
    pl.kernel,
    mesh=_sc_mesh,
    out_type=jax.ShapeDtypeStruct((_N, _D), jnp.float32),
    scratch_types=[
        pltpu.VMEM((_PER_W,), jnp.int32),
        pltpu.VMEM((_CH, _D), jnp.float32),
        pltpu.VMEM((_CH, _D), jnp.float32),
        pltpu.SemaphoreType.DMA,
        pltpu.SemaphoreType.DMA,
        pltpu.SemaphoreType.DMA,
        pltpu.SemaphoreType.DMA,
    ],
)
def _sc_gather(ids_hbm, table_hbm, out_hbm, idx_v, rows_a, rows_b,
               sga, sgb, swa, swb):
    wid = lax.axis_index("s") * _NC + lax.axis_index("c")
    obase = pl.multiple_of(wid * _PER_W, _CH)

    pltpu.sync_copy(ids_hbm.at[pl.ds(obase, _PER_W)], idx_v)

    def idx_at(c):
        return idx_v.at[pl.ds(pl.multiple_of(c * _CH, _CH), _CH)]

    def out_at(c):
        return out_hbm.at[pl.ds(obase + pl.multiple_of(c * _CH, _CH), _CH)]

    def g_start(c, buf, sem):
        pltpu.async_copy(table_hbm.at[idx_at(c)], buf, sem)

    def g_wait(c, buf, sem):
        pltpu.make_async_copy(table_hbm.at[idx_at(c)], buf, sem).wait()

    def w_start(c, buf, sem):
        pltpu.async_copy(buf, out_at(c), sem)

    def w_wait(c, buf, sem):
        pltpu.make_async_copy(buf, out_at(c), sem).wait()

    g_start(0, rows_a, sga)

    def body(i, carry):
        c0 = 2 * i
        g_wait(c0, rows_a, sga)

        @pl.when(i > 0)
        def _():
            w_wait(c0 - 1, rows_b, swb)

        g_start(c0 + 1, rows_b, sgb)
        w_start(c0, rows_a, swa)
        g_wait(c0 + 1, rows_b, sgb)
        w_wait(c0, rows_a, swa)

        @pl.when(i < _NCH // 2 - 1)
        def _():
            g_start(c0 + 2, rows_a, sga)

        w_start(c0 + 1, rows_b, swb)
        return carry

    lax.fori_loop(0, _NCH // 2, body, 0)
    w_wait(_NCH - 1, rows_b, swb)


def kernel(input_, W0, W1, W2, P0, P1, P2):
    table = _build_table(W0, W1, W2, P0, P1, P2)
    ids = input_.reshape(_N)
    out = _sc_gather(ids, table)
    return out.reshape(_BATCH, _SEQ, _D)


# speedup vs baseline: 1.4037x; 1.0942x over previous
import functools

import jax
import jax.numpy as jnp
from jax import lax
from jax.experimental import pallas as pl
from jax.experimental.pallas import tpu as pltpu
from jax.experimental.pallas import tpu_sc as plsc

_D = 128
_SCALE = float(_D) ** 0.5
_BATCH, _SEQ = 4096, 200
_NUM_TOKENS = 1000000

_R = 20000
_NSTEPS = _NUM_TOKENS // _R
_S0 = 20000 // _R
_S1 = 100000 // _R


def _table_body(w0, w1, w2, q0, q1, q2, out):
    s = pl.program_id(0)
    dn = (((1,), (0,)), ((), ()))

    @pl.when(s < _S0)
    def _():
        out[...] = lax.dot_general(
            w0[...], q0[...], dn, preferred_element_type=jnp.float32)

    @pl.when((s >= _S0) & (s < _S1))
    def _():
        out[...] = lax.dot_general(
            w1[...], q1[...], dn, preferred_element_type=jnp.float32)

    @pl.when(s >= _S1)
    def _():
        out[...] = lax.dot_general(
            w2[...], q2[...], dn, preferred_element_type=jnp.float32)


def _build_table(W0, W1, W2, P0, P1, P2):
    bf = jnp.bfloat16
    return pl.pallas_call(
        _table_body,
        grid=(_NSTEPS,),
        in_specs=[
            pl.BlockSpec((_R, 128), lambda s: (jnp.where(s < _S0, s, 0), 0)),
            pl.BlockSpec((_R, 32),
                         lambda s: (jnp.clip(s - _S0, 0, _S1 - _S0 - 1), 0)),
            pl.BlockSpec((_R, 8),
                         lambda s: (jnp.clip(s - _S1, 0,
                                             _NSTEPS - _S1 - 1), 0)),
            pl.BlockSpec((128, 128), lambda s: (0, 0)),
            pl.BlockSpec((32, 128), lambda s: (0, 0)),
            pl.BlockSpec((8, 128), lambda s: (0, 0)),
        ],
        out_specs=pl.BlockSpec((_R, _D), lambda s: (s, 0)),
        out_shape=jax.ShapeDtypeStruct((_NUM_TOKENS, _D), jnp.float32),
    )(W0.astype(bf), W1.astype(bf), W2.astype(bf),
      (P0.T * _SCALE).astype(bf), (P1.T * _SCALE).astype(bf),
      (P2.T * _SCALE).astype(bf))


_N = _BATCH * _SEQ
_NC, _NS = 2, 16
_NW = _NC * _NS
_PER_W = _N // _NW
_CH = 128
_NCH = _PER_W // _CH

_sc_mesh = plsc.VectorSubcoreMesh(core_axis_name="c", subcore_axis_name="s")


@functools.partial(
    pl.kernel,
    mesh=_sc_mesh,
    out_type=jax.ShapeDtypeStruct((_N, _D), jnp.float32),
    scratch_types=[
        pltpu.VMEM((_PER_W,), jnp.int32),
        pltpu.VMEM((_CH, _D), jnp.float32),
        pltpu.VMEM((_CH, _D), jnp.float32),
        pltpu.VMEM((_CH, _D), jnp.float32),
        pltpu.VMEM((_CH, _D), jnp.float32),
        pltpu.SemaphoreType.DMA,
        pltpu.SemaphoreType.DMA,
        pltpu.SemaphoreType.DMA,
        pltpu.SemaphoreType.DMA,
        pltpu.SemaphoreType.DMA,
        pltpu.SemaphoreType.DMA,
        pltpu.SemaphoreType.DMA,
        pltpu.SemaphoreType.DMA,
    ],
)
def _sc_gather(ids_hbm, table_hbm, out_hbm, idx_v, r0, r1, r2, r3,
               g0, g1, g2, g3, w0, w1, w2, w3):
    wid = lax.axis_index("s") * _NC + lax.axis_index("c")
    obase = pl.multiple_of(wid * _PER_W, _CH)
    rbuf = (r0, r1, r2, r3)
    gsem = (g0, g1, g2, g3)
    wsem = (w0, w1, w2, w3)

    pltpu.sync_copy(ids_hbm.at[pl.ds(obase, _PER_W)], idx_v)

    def idx_at(c):
        return idx_v.at[pl.ds(pl.multiple_of(c * _CH, _CH), _CH)]

    def out_at(c):
        return out_hbm.at[pl.ds(obase + pl.multiple_of(c * _CH, _CH), _CH)]

    def g_start(c, buf, sem):
        pltpu.async_copy(table_hbm.at[idx_at(c)], buf, sem)

    def g_wait(c, buf, sem):
        pltpu.make_async_copy(table_hbm.at[idx_at(c)], buf, sem).wait()

    def w_start(c, buf, sem):
        pltpu.async_copy(buf, out_at(c), sem)

    def w_wait(c, buf, sem):
        pltpu.make_async_copy(buf, out_at(c), sem).wait()

    for j in range(3):
        g_start(j, rbuf[j], gsem[j])

    def body(i, carry):
        for j in range(4):
            c = 4 * i + j
            jp = (j + 3) % 4
            g_wait(c, rbuf[j], gsem[j])
            w_start(c, rbuf[j], wsem[j])
            if j == 0:
                @pl.when(i > 0)
                def _(c=c, jp=jp):
                    w_wait(c - 1, rbuf[jp], wsem[jp])
            else:
                w_wait(c - 1, rbuf[jp], wsem[jp])

            @pl.when(c + 3 < _NCH)
            def _(c=c, jp=jp):
                g_start(c + 3, rbuf[jp], gsem[jp])
        return carry

    lax.fori_loop(0, _NCH // 4, body, 0)
    w_wait(_NCH - 1, rbuf[3], wsem[3])


def kernel(input_, W0, W1, W2, P0, P1, P2):
    table = _build_table(W0, W1, W2, P0, P1, P2)
    ids = input_.reshape(_N)
    out = _sc_gather(ids, table)
    return out.reshape(_BATCH, _SEQ, _D)
